# baseline (speedup 1.0000x reference)
import functools

import jax
import jax.numpy as jnp
from jax import lax
from jax.experimental import pallas as pl
from jax.experimental.pallas import tpu as pltpu
from jax.experimental.pallas import tpu_sc as plsc

N = 10000
E = 320000
C = 128
NC = 2
NS = 16
K = 128
NCHUNK = 80
SCHUNK = 8
GH = 4
GS = K // GH
EPT = K * NCHUNK
EPAD = EPT * NC * NS
ACC_ROWS = 10240
RPT = ACC_ROWS // NS
RB = 1000
RB2 = 1024


def _dense_a_body(x_ref, we_ref, be_ref, w1_ref, b1_ref, out_ref):
    h = jnp.dot(x_ref[...], we_ref[...], preferred_element_type=jnp.float32)
    h = h + be_ref[...]
    out_ref[...] = jax.nn.gelu(
        jnp.dot(h, w1_ref[...], preferred_element_type=jnp.float32) + b1_ref[...])


def _dense_b_body(sums_ref, cnt_ref, w2_ref, b2_ref, out_ref):
    p = sums_ref[0] + sums_ref[1]
    mean = p / jnp.maximum(cnt_ref[...], 1.0)
    upd = jax.nn.gelu(jnp.dot(mean, w2_ref[...], preferred_element_type=jnp.float32)
                      + b2_ref[...])
    out_ref[...] = jnp.mean(upd, axis=1, keepdims=True)


def _zero_buf(buf):
    zv = jnp.zeros((16,), jnp.float32)

    def zb(i, carry):
        buf[i // (C // 16), pl.ds((i % (C // 16)) * 16, 16)] = zv
        return carry

    lax.fori_loop(0, K * (C // 16), zb, 0)


def _zero_acc_slice(buf, acc, s):
    def za(j, carry):
        pltpu.sync_copy(buf, acc.at[pl.ds(s * RPT + j * K, K), :])
        return carry

    lax.fori_loop(0, RPT // K, za, 0)


def _write_out_slice(buf, acc, out, c, s):
    def wout(j, carry):
        r = s * RPT + j * K
        pltpu.sync_copy(acc.at[pl.ds(r, K), :], buf)
        pltpu.sync_copy(buf, out.at[c, pl.ds(r, K), :])
        return carry

    lax.fori_loop(0, RPT // K, wout, 0)


def _sc_sum_body(g_hbm, src_hbm, dst_hbm, out_sum,
                 src_v, dst_v, buf0, buf1, acc, semg0, semg1, sems0, sems1):
    c = lax.axis_index("c")
    s = lax.axis_index("s")

    _zero_buf(buf0)
    _zero_acc_slice(buf0, acc, s)
    plsc.subcore_barrier()

    bufs = (buf0, buf1)
    semg = (semg0, semg1)
    sems = (sems0, sems1)

    def stage(t, carry):
        pltpu.sync_copy(src_hbm.at[c, s, pl.ds(t * SCHUNK, SCHUNK)], src_v)
        pltpu.sync_copy(dst_hbm.at[c, s, pl.ds(t * SCHUNK, SCHUNK)], dst_v)
        gd = [None, None]
        sd = [None, None]

        def gather(j, p):
            return [pltpu.async_copy(
                        g_hbm.at[src_v.at[j, pl.ds(h * GS, GS)]],
                        bufs[p].at[pl.ds(h * GS, GS), :], semg[p])
                    for h in range(GH)]

        for j in range(SCHUNK):
            p = j & 1
            if sd[p] is not None:
                sd[p].wait()
            gd[p] = gather(j, p)
            if j >= 1:
                q = 1 - p
                for d in gd[q]:
                    d.wait()
                sd[q] = pltpu.async_copy(bufs[q], acc.at[dst_v.at[j - 1]],
                                         sems[q], add=True)
        q = (SCHUNK - 1) & 1
        for d in gd[q]:
            d.wait()
        sd[q] = pltpu.async_copy(bufs[q], acc.at[dst_v.at[SCHUNK - 1]],
                                 sems[q], add=True)
        for p in range(2):
            if sd[p] is not None:
                sd[p].wait()
        return carry

    lax.fori_loop(0, NCHUNK // SCHUNK, stage, 0)
    plsc.subcore_barrier()
    _write_out_slice(buf0, acc, out_sum, c, s)


def _sc_cnt_body(dst_hbm, out_cnt, dst_v, ones_v, cac, sem):
    c = lax.axis_index("c")
    s = lax.axis_index("s")

    _zero_buf(ones_v)
    _zero_acc_slice(ones_v, cac, s)

    ov = jnp.ones((16,), jnp.float32)

    def fo(i, carry):
        ones_v[i // (C // 16), pl.ds((i % (C // 16)) * 16, 16)] = ov
        return carry

    lax.fori_loop(0, K * (C // 16), fo, 0)
    plsc.subcore_barrier()

    def stage(t, carry):
        pltpu.sync_copy(dst_hbm.at[c, s, pl.ds(t * SCHUNK, SCHUNK)], dst_v)
        sds = [pltpu.async_copy(ones_v, cac.at[dst_v.at[j]], sem, add=True)
               for j in range(SCHUNK)]
        for sd in sds:
            sd.wait()
        return carry

    lax.fori_loop(0, NCHUNK // SCHUNK, stage, 0)
    plsc.subcore_barrier()
    _write_out_slice(ones_v, cac, out_cnt, c, s)


def _make_sc_sum():
    return functools.partial(
        pl.kernel,
        out_type=jax.ShapeDtypeStruct((NC, ACC_ROWS, C), jnp.float32),
        mesh=plsc.VectorSubcoreMesh(core_axis_name="c", subcore_axis_name="s",
                                    num_cores=NC, num_subcores=NS),
        scratch_types=[
            pltpu.VMEM((SCHUNK, K), jnp.int32),
            pltpu.VMEM((SCHUNK, K), jnp.int32),
            pltpu.VMEM((K, C), jnp.float32),
            pltpu.VMEM((K, C), jnp.float32),
            pltpu.VMEM_SHARED((ACC_ROWS, C), jnp.float32),
            pltpu.SemaphoreType.DMA,
            pltpu.SemaphoreType.DMA,
            pltpu.SemaphoreType.DMA,
            pltpu.SemaphoreType.DMA,
        ],
    )(_sc_sum_body)


def _make_sc_cnt():
    return functools.partial(
        pl.kernel,
        out_type=jax.ShapeDtypeStruct((NC, ACC_ROWS, C), jnp.float32),
        mesh=plsc.VectorSubcoreMesh(core_axis_name="c", subcore_axis_name="s",
                                    num_cores=NC, num_subcores=NS),
        scratch_types=[
            pltpu.VMEM((SCHUNK, K), jnp.int32),
            pltpu.VMEM((K, C), jnp.float32),
            pltpu.VMEM_SHARED((ACC_ROWS, C), jnp.float32),
            pltpu.SemaphoreType.DMA,
        ],
    )(_sc_cnt_body)


def kernel(x, edge_index, W_embed, b_embed, W1, b1, W2, b2):
    x = x.astype(jnp.float32)

    g = pl.pallas_call(
        _dense_a_body,
        grid=(N // RB,),
        in_specs=[
            pl.BlockSpec((RB, x.shape[1]), lambda i: (i, 0)),
            pl.BlockSpec(W_embed.shape, lambda i: (0, 0)),
            pl.BlockSpec((1, C), lambda i: (0, 0)),
            pl.BlockSpec(W1.shape, lambda i: (0, 0)),
            pl.BlockSpec((1, C), lambda i: (0, 0)),
        ],
        out_specs=pl.BlockSpec((RB, C), lambda i: (i, 0)),
        out_shape=jax.ShapeDtypeStruct((N, C), jnp.float32),
    )(x, W_embed, b_embed.reshape(1, C), W1, b1.reshape(1, C))

    ei = edge_index.astype(jnp.int32)
    pad = EPAD - E
    src_p = jnp.concatenate([ei[0], jnp.zeros((pad,), jnp.int32)])
    dst_p = jnp.concatenate([ei[1], jnp.full((pad,), N, jnp.int32)])
    src_r = src_p.reshape(NC, NS, NCHUNK, K)
    dst_r = dst_p.reshape(NC, NS, NCHUNK, K)

    sums = _make_sc_sum()(g, src_r, dst_r)
    cnts = _make_sc_cnt()(dst_r)

    cnt = cnts.sum(axis=0)[:, 0:1]

    out2d = pl.pallas_call(
        _dense_b_body,
        grid=(ACC_ROWS // RB2,),
        in_specs=[
            pl.BlockSpec((NC, RB2, C), lambda i: (0, i, 0)),
            pl.BlockSpec((RB2, 1), lambda i: (i, 0)),
            pl.BlockSpec(W2.shape, lambda i: (0, 0)),
            pl.BlockSpec((1, C), lambda i: (0, 0)),
        ],
        out_specs=pl.BlockSpec((RB2, 1), lambda i: (i, 0)),
        out_shape=jax.ShapeDtypeStruct((ACC_ROWS, 1), jnp.float32),
    )(sums, cnt, W2, b2.reshape(1, C))

    return out2d.reshape(ACC_ROWS)[:N]


# speedup vs baseline: 1.1716x; 1.1716x over previous
import functools

import jax
import jax.numpy as jnp
from jax import lax
from jax.experimental import pallas as pl
from jax.experimental.pallas import tpu as pltpu
from jax.experimental.pallas import tpu_sc as plsc

N = 10000
E = 320000
C = 128
NC = 2
NS = 16
K = 128
NCHUNK = 80
SCHUNK = 8
GH = 4
GS = K // GH
EPT = K * NCHUNK
EPAD = EPT * NC * NS
ACC_ROWS = 10240
RPT = ACC_ROWS // NS
RB = 1000
RB2 = 1024


def _dense_a_body(x_ref, we_ref, be_ref, w1_ref, b1_ref, out_ref):
    h = jnp.dot(x_ref[...], we_ref[...], preferred_element_type=jnp.float32)
    h = h + be_ref[...]
    out_ref[...] = jax.nn.gelu(
        jnp.dot(h, w1_ref[...], preferred_element_type=jnp.float32) + b1_ref[...])


def _dense_b_body(sums_ref, cnt_ref, w2_ref, b2_ref, out_ref):
    p = sums_ref[0] + sums_ref[1]
    mean = p / jnp.maximum(cnt_ref[...], 1.0)
    upd = jax.nn.gelu(jnp.dot(mean, w2_ref[...], preferred_element_type=jnp.float32)
                      + b2_ref[...])
    out_ref[...] = jnp.mean(upd, axis=1, keepdims=True)


def _zero_buf(buf):
    zv = jnp.zeros((16,), jnp.float32)

    def zb(i, carry):
        buf[i // (C // 16), pl.ds((i % (C // 16)) * 16, 16)] = zv
        return carry

    lax.fori_loop(0, K * (C // 16), zb, 0)


def _zero_acc_slice(buf, acc, s):
    def za(j, carry):
        pltpu.sync_copy(buf, acc.at[pl.ds(s * RPT + j * K, K), :])
        return carry

    lax.fori_loop(0, RPT // K, za, 0)


def _write_out_slice(buf, acc, out, c, s):
    def wout(j, carry):
        r = s * RPT + j * K
        pltpu.sync_copy(acc.at[pl.ds(r, K), :], buf)
        pltpu.sync_copy(buf, out.at[c, pl.ds(r, K), :])
        return carry

    lax.fori_loop(0, RPT // K, wout, 0)


def _sc_sum_body(g_hbm, src_hbm, dst_hbm, out_sum,
                 src_v, dst_v, buf0, buf1, acc, semg0, semg1, sems0, sems1):
    c = lax.axis_index("c")
    s = lax.axis_index("s")

    _zero_buf(buf0)
    _zero_acc_slice(buf0, acc, s)
    plsc.subcore_barrier()

    bufs = (buf0, buf1)
    semg = (semg0, semg1)
    sems = (sems0, sems1)

    def stage(t, carry):
        pltpu.sync_copy(src_hbm.at[c, s, pl.ds(t * SCHUNK, SCHUNK)], src_v)
        pltpu.sync_copy(dst_hbm.at[c, s, pl.ds(t * SCHUNK, SCHUNK)], dst_v)
        gd = [None, None]
        sd = [None, None]

        def gather(j, p):
            return [pltpu.async_copy(
                        g_hbm.at[src_v.at[j, pl.ds(h * GS, GS)]],
                        bufs[p].at[pl.ds(h * GS, GS), :], semg[p])
                    for h in range(GH)]

        for j in range(SCHUNK):
            p = j & 1
            if sd[p] is not None:
                sd[p].wait()
            gd[p] = gather(j, p)
            if j >= 1:
                q = 1 - p
                for d in gd[q]:
                    d.wait()
                sd[q] = pltpu.async_copy(bufs[q], acc.at[dst_v.at[j - 1]],
                                         sems[q], add=True)
        q = (SCHUNK - 1) & 1
        for d in gd[q]:
            d.wait()
        sd[q] = pltpu.async_copy(bufs[q], acc.at[dst_v.at[SCHUNK - 1]],
                                 sems[q], add=True)
        for p in range(2):
            if sd[p] is not None:
                sd[p].wait()
        return carry

    lax.fori_loop(0, NCHUNK // SCHUNK, stage, 0)
    plsc.subcore_barrier()
    _write_out_slice(buf0, acc, out_sum, c, s)


def _sc_cnt_body(dst_hbm, out_cnt, dst_v, ones_v, cac, sem):
    c = lax.axis_index("c")
    s = lax.axis_index("s")

    _zero_buf(ones_v)
    _zero_acc_slice(ones_v, cac, s)

    ov = jnp.ones((16,), jnp.float32)

    def fo(i, carry):
        ones_v[i // (C // 16), pl.ds((i % (C // 16)) * 16, 16)] = ov
        return carry

    lax.fori_loop(0, K * (C // 16), fo, 0)
    plsc.subcore_barrier()

    def stage(t, carry):
        pltpu.sync_copy(dst_hbm.at[c, s, pl.ds(t * SCHUNK, SCHUNK)], dst_v)
        sds = [pltpu.async_copy(ones_v, cac.at[dst_v.at[j]], sem, add=True)
               for j in range(SCHUNK)]
        for sd in sds:
            sd.wait()
        return carry

    lax.fori_loop(0, NCHUNK // SCHUNK, stage, 0)
    plsc.subcore_barrier()
    _write_out_slice(ones_v, cac, out_cnt, c, s)


def _make_sc_sum():
    return functools.partial(
        pl.kernel,
        out_type=jax.ShapeDtypeStruct((NC, ACC_ROWS, C), jnp.float32),
        mesh=plsc.VectorSubcoreMesh(core_axis_name="c", subcore_axis_name="s",
                                    num_cores=NC, num_subcores=NS),
        scratch_types=[
            pltpu.VMEM((SCHUNK, K), jnp.int32),
            pltpu.VMEM((SCHUNK, K), jnp.int32),
            pltpu.VMEM((K, C), jnp.float32),
            pltpu.VMEM((K, C), jnp.float32),
            pltpu.VMEM_SHARED((ACC_ROWS, C), jnp.float32),
            pltpu.SemaphoreType.DMA,
            pltpu.SemaphoreType.DMA,
            pltpu.SemaphoreType.DMA,
            pltpu.SemaphoreType.DMA,
        ],
    )(_sc_sum_body)


def _make_sc_cnt():
    return functools.partial(
        pl.kernel,
        out_type=jax.ShapeDtypeStruct((NC, ACC_ROWS, C), jnp.float32),
        mesh=plsc.VectorSubcoreMesh(core_axis_name="c", subcore_axis_name="s",
                                    num_cores=NC, num_subcores=NS),
        scratch_types=[
            pltpu.VMEM((SCHUNK, K), jnp.int32),
            pltpu.VMEM((K, C), jnp.float32),
            pltpu.VMEM_SHARED((ACC_ROWS, C), jnp.float32),
            pltpu.SemaphoreType.DMA,
        ],
    )(_sc_cnt_body)


def kernel(x, edge_index, W_embed, b_embed, W1, b1, W2, b2):
    x = x.astype(jnp.float32)

    g = pl.pallas_call(
        _dense_a_body,
        grid=(N // RB,),
        in_specs=[
            pl.BlockSpec((RB, x.shape[1]), lambda i: (i, 0)),
            pl.BlockSpec(W_embed.shape, lambda i: (0, 0)),
            pl.BlockSpec((1, C), lambda i: (0, 0)),
            pl.BlockSpec(W1.shape, lambda i: (0, 0)),
            pl.BlockSpec((1, C), lambda i: (0, 0)),
        ],
        out_specs=pl.BlockSpec((RB, C), lambda i: (i, 0)),
        out_shape=jax.ShapeDtypeStruct((N, C), jnp.float32),
    )(x, W_embed, b_embed.reshape(1, C), W1, b1.reshape(1, C))

    ei = edge_index.astype(jnp.int32)
    pad = EPAD - E
    src_p = jnp.concatenate([ei[0], jnp.zeros((pad,), jnp.int32)])
    dst_p = jnp.concatenate([ei[1], jnp.full((pad,), N, jnp.int32)])
    src_r = src_p.reshape(NC, NS, NCHUNK, K)
    dst_r = dst_p.reshape(NC, NS, NCHUNK, K)

    g2 = jnp.concatenate([g, g], axis=0)
    src_r = src_r + (jnp.arange(NC, dtype=jnp.int32) * N).reshape(NC, 1, 1, 1)

    sums = _make_sc_sum()(g2, src_r, dst_r)
    cnts = _make_sc_cnt()(dst_r)

    cnt = cnts.sum(axis=0)[:, 0:1]

    out2d = pl.pallas_call(
        _dense_b_body,
        grid=(ACC_ROWS // RB2,),
        in_specs=[
            pl.BlockSpec((NC, RB2, C), lambda i: (0, i, 0)),
            pl.BlockSpec((RB2, 1), lambda i: (i, 0)),
            pl.BlockSpec(W2.shape, lambda i: (0, 0)),
            pl.BlockSpec((1, C), lambda i: (0, 0)),
        ],
        out_specs=pl.BlockSpec((RB2, 1), lambda i: (i, 0)),
        out_shape=jax.ShapeDtypeStruct((ACC_ROWS, 1), jnp.float32),
    )(sums, cnt, W2, b2.reshape(1, C))

    return out2d.reshape(ACC_ROWS)[:N]


# speedup vs baseline: 1.1927x; 1.0180x over previous
import functools

import jax
import jax.numpy as jnp
from jax import lax
from jax.experimental import pallas as pl
from jax.experimental.pallas import tpu as pltpu
from jax.experimental.pallas import tpu_sc as plsc

N = 10000
E = 320000
C = 128
NC = 2
NS = 16
K = 128
NCHUNK = 80
SCHUNK = 8
GH = 4
GS = K // GH
NCH0 = 48
NCH1 = 112
NCHMAX = max(NCH0, NCH1)
EPT = K * NCHUNK
EPAD = EPT * NC * NS
ACC_ROWS = 10240
RPT = ACC_ROWS // NS
RB = 1000
RB2 = 1024


def _dense_a_body(x_ref, we_ref, be_ref, w1_ref, b1_ref, out_ref):
    h = jnp.dot(x_ref[...], we_ref[...], preferred_element_type=jnp.float32)
    h = h + be_ref[...]
    out_ref[...] = jax.nn.gelu(
        jnp.dot(h, w1_ref[...], preferred_element_type=jnp.float32) + b1_ref[...])


def _dense_b_body(sums_ref, cnt_ref, w2_ref, b2_ref, out_ref):
    p = sums_ref[0] + sums_ref[1]
    mean = p / jnp.maximum(cnt_ref[...], 1.0)
    upd = jax.nn.gelu(jnp.dot(mean, w2_ref[...], preferred_element_type=jnp.float32)
                      + b2_ref[...])
    out_ref[...] = jnp.mean(upd, axis=1, keepdims=True)


def _zero_buf(buf):
    zv = jnp.zeros((16,), jnp.float32)

    def zb(i, carry):
        buf[i // (C // 16), pl.ds((i % (C // 16)) * 16, 16)] = zv
        return carry

    lax.fori_loop(0, K * (C // 16), zb, 0)


def _zero_acc_slice(buf, acc, s):
    def za(j, carry):
        pltpu.sync_copy(buf, acc.at[pl.ds(s * RPT + j * K, K), :])
        return carry

    lax.fori_loop(0, RPT // K, za, 0)


def _write_out_slice(buf, acc, out, c, s):
    def wout(j, carry):
        r = s * RPT + j * K
        pltpu.sync_copy(acc.at[pl.ds(r, K), :], buf)
        pltpu.sync_copy(buf, out.at[c, pl.ds(r, K), :])
        return carry

    lax.fori_loop(0, RPT // K, wout, 0)


def _sc_sum_body(g_hbm, src_hbm, dst_hbm, out_sum,
                 src_v, dst_v, buf0, buf1, acc, semg0, semg1, sems0, sems1):
    c = lax.axis_index("c")
    s = lax.axis_index("s")

    _zero_buf(buf0)
    _zero_acc_slice(buf0, acc, s)
    plsc.subcore_barrier()

    bufs = (buf0, buf1)
    semg = (semg0, semg1)
    sems = (sems0, sems1)

    def stage(t, carry):
        pltpu.sync_copy(src_hbm.at[c, s, pl.ds(t * SCHUNK, SCHUNK)], src_v)
        pltpu.sync_copy(dst_hbm.at[c, s, pl.ds(t * SCHUNK, SCHUNK)], dst_v)
        gd = [None, None]
        sd = [None, None]

        def gather(j, p):
            return [pltpu.async_copy(
                        g_hbm.at[src_v.at[j, pl.ds(h * GS, GS)]],
                        bufs[p].at[pl.ds(h * GS, GS), :], semg[p])
                    for h in range(GH)]

        for j in range(SCHUNK):
            p = j & 1
            if sd[p] is not None:
                sd[p].wait()
            gd[p] = gather(j, p)
            if j >= 1:
                q = 1 - p
                for d in gd[q]:
                    d.wait()
                sd[q] = pltpu.async_copy(bufs[q], acc.at[dst_v.at[j - 1]],
                                         sems[q], add=True)
        q = (SCHUNK - 1) & 1
        for d in gd[q]:
            d.wait()
        sd[q] = pltpu.async_copy(bufs[q], acc.at[dst_v.at[SCHUNK - 1]],
                                 sems[q], add=True)
        for p in range(2):
            if sd[p] is not None:
                sd[p].wait()
        return carry

    nstage = jnp.where(c == 0, NCH0 // SCHUNK, NCH1 // SCHUNK)
    lax.fori_loop(0, nstage, stage, 0)
    plsc.subcore_barrier()
    _write_out_slice(buf0, acc, out_sum, c, s)


def _sc_cnt_body(dst_hbm, out_cnt, dst_v, ones_v, cac, sem):
    c = lax.axis_index("c")
    s = lax.axis_index("s")

    _zero_buf(ones_v)
    _zero_acc_slice(ones_v, cac, s)

    ov = jnp.ones((16,), jnp.float32)

    def fo(i, carry):
        ones_v[i // (C // 16), pl.ds((i % (C // 16)) * 16, 16)] = ov
        return carry

    lax.fori_loop(0, K * (C // 16), fo, 0)
    plsc.subcore_barrier()

    def stage(t, carry):
        pltpu.sync_copy(dst_hbm.at[c, s, pl.ds(t * SCHUNK, SCHUNK)], dst_v)
        sds = [pltpu.async_copy(ones_v, cac.at[dst_v.at[j]], sem, add=True)
               for j in range(SCHUNK)]
        for sd in sds:
            sd.wait()
        return carry

    lax.fori_loop(0, NCHUNK // SCHUNK, stage, 0)
    plsc.subcore_barrier()
    _write_out_slice(ones_v, cac, out_cnt, c, s)


def _make_sc_sum():
    return functools.partial(
        pl.kernel,
        out_type=jax.ShapeDtypeStruct((NC, ACC_ROWS, C), jnp.float32),
        mesh=plsc.VectorSubcoreMesh(core_axis_name="c", subcore_axis_name="s",
                                    num_cores=NC, num_subcores=NS),
        scratch_types=[
            pltpu.VMEM((SCHUNK, K), jnp.int32),
            pltpu.VMEM((SCHUNK, K), jnp.int32),
            pltpu.VMEM((K, C), jnp.float32),
            pltpu.VMEM((K, C), jnp.float32),
            pltpu.VMEM_SHARED((ACC_ROWS, C), jnp.float32),
            pltpu.SemaphoreType.DMA,
            pltpu.SemaphoreType.DMA,
            pltpu.SemaphoreType.DMA,
            pltpu.SemaphoreType.DMA,
        ],
    )(_sc_sum_body)


def _make_sc_cnt():
    return functools.partial(
        pl.kernel,
        out_type=jax.ShapeDtypeStruct((NC, ACC_ROWS, C), jnp.float32),
        mesh=plsc.VectorSubcoreMesh(core_axis_name="c", subcore_axis_name="s",
                                    num_cores=NC, num_subcores=NS),
        scratch_types=[
            pltpu.VMEM((SCHUNK, K), jnp.int32),
            pltpu.VMEM((K, C), jnp.float32),
            pltpu.VMEM_SHARED((ACC_ROWS, C), jnp.float32),
            pltpu.SemaphoreType.DMA,
        ],
    )(_sc_cnt_body)


def kernel(x, edge_index, W_embed, b_embed, W1, b1, W2, b2):
    x = x.astype(jnp.float32)

    g = pl.pallas_call(
        _dense_a_body,
        grid=(N // RB,),
        in_specs=[
            pl.BlockSpec((RB, x.shape[1]), lambda i: (i, 0)),
            pl.BlockSpec(W_embed.shape, lambda i: (0, 0)),
            pl.BlockSpec((1, C), lambda i: (0, 0)),
            pl.BlockSpec(W1.shape, lambda i: (0, 0)),
            pl.BlockSpec((1, C), lambda i: (0, 0)),
        ],
        out_specs=pl.BlockSpec((RB, C), lambda i: (i, 0)),
        out_shape=jax.ShapeDtypeStruct((N, C), jnp.float32),
    )(x, W_embed, b_embed.reshape(1, C), W1, b1.reshape(1, C))

    ei = edge_index.astype(jnp.int32)
    pad = EPAD - E
    src_p = jnp.concatenate([ei[0], jnp.zeros((pad,), jnp.int32)])
    dst_p = jnp.concatenate([ei[1], jnp.full((pad,), N, jnp.int32)])
    dst_r = dst_p.reshape(NC, NS, NCHUNK, K)

    e0 = NS * NCH0 * K
    cpad = ((0, 0), (0, 0), (0, NCHMAX - NCH0), (0, 0))
    src_a = jnp.concatenate(
        [jnp.pad(src_p[:e0].reshape(1, NS, NCH0, K), cpad),
         src_p[e0:].reshape(1, NS, NCH1, K)], axis=0)
    dst_a = jnp.concatenate(
        [jnp.pad(dst_p[:e0].reshape(1, NS, NCH0, K), cpad, constant_values=N),
         dst_p[e0:].reshape(1, NS, NCH1, K)], axis=0)

    g2 = jnp.concatenate([g, g], axis=0)
    src_a = src_a + (jnp.arange(NC, dtype=jnp.int32) * N).reshape(NC, 1, 1, 1)

    sums = _make_sc_sum()(g2, src_a, dst_a)
    cnts = _make_sc_cnt()(dst_r)

    cnt = cnts.sum(axis=0)[:, 0:1]

    out2d = pl.pallas_call(
        _dense_b_body,
        grid=(ACC_ROWS // RB2,),
        in_specs=[
            pl.BlockSpec((NC, RB2, C), lambda i: (0, i, 0)),
            pl.BlockSpec((RB2, 1), lambda i: (i, 0)),
            pl.BlockSpec(W2.shape, lambda i: (0, 0)),
            pl.BlockSpec((1, C), lambda i: (0, 0)),
        ],
        out_specs=pl.BlockSpec((RB2, 1), lambda i: (i, 0)),
        out_shape=jax.ShapeDtypeStruct((ACC_ROWS, 1), jnp.float32),
    )(sums, cnt, W2, b2.reshape(1, C))

    return out2d.reshape(ACC_ROWS)[:N]


# speedup vs baseline: 1.2262x; 1.0281x over previous
import functools

import jax
import jax.numpy as jnp
from jax import lax
from jax.experimental import pallas as pl
from jax.experimental.pallas import tpu as pltpu
from jax.experimental.pallas import tpu_sc as plsc

N = 10000
E = 320000
C = 128
NC = 2
NS = 16
K = 128
NCHUNK = 80
SCHUNK = 16
GH = 4
GS = K // GH
NCH0 = 48
NCH1 = 112
NCHMAX = max(NCH0, NCH1)
EPT = K * NCHUNK
EPAD = EPT * NC * NS
ACC_ROWS = 10240
RPT = ACC_ROWS // NS
RB = 1000
RB2 = 1024


def _dense_a_body(x_ref, we_ref, be_ref, w1_ref, b1_ref, out_ref):
    h = jnp.dot(x_ref[...], we_ref[...], preferred_element_type=jnp.float32)
    h = h + be_ref[...]
    out_ref[...] = jax.nn.gelu(
        jnp.dot(h, w1_ref[...], preferred_element_type=jnp.float32) + b1_ref[...])


def _dense_b_body(sums_ref, cnt_ref, w2_ref, b2_ref, out_ref):
    p = sums_ref[0] + sums_ref[1]
    mean = p / jnp.maximum(cnt_ref[...], 1.0)
    upd = jax.nn.gelu(jnp.dot(mean, w2_ref[...], preferred_element_type=jnp.float32)
                      + b2_ref[...])
    out_ref[...] = jnp.mean(upd, axis=1, keepdims=True)


def _zero_buf(buf):
    zv = jnp.zeros((16,), jnp.float32)

    def zb(i, carry):
        buf[i // (C // 16), pl.ds((i % (C // 16)) * 16, 16)] = zv
        return carry

    lax.fori_loop(0, K * (C // 16), zb, 0)


def _zero_acc_slice(buf, acc, s):
    def za(j, carry):
        pltpu.sync_copy(buf, acc.at[pl.ds(s * RPT + j * K, K), :])
        return carry

    lax.fori_loop(0, RPT // K, za, 0)


def _write_out_slice(buf, acc, out, c, s):
    def wout(j, carry):
        r = s * RPT + j * K
        pltpu.sync_copy(acc.at[pl.ds(r, K), :], buf)
        pltpu.sync_copy(buf, out.at[c, pl.ds(r, K), :])
        return carry

    lax.fori_loop(0, RPT // K, wout, 0)


def _sc_sum_body(g_hbm, src_hbm, dst_hbm, out_sum,
                 src_v, dst_v, buf0, buf1, acc, semg0, semg1, sems0, sems1):
    c = lax.axis_index("c")
    s = lax.axis_index("s")

    _zero_buf(buf0)
    _zero_acc_slice(buf0, acc, s)
    plsc.subcore_barrier()

    bufs = (buf0, buf1)
    semg = (semg0, semg1)
    sems = (sems0, sems1)

    def stage(t, carry):
        pltpu.sync_copy(src_hbm.at[c, s, pl.ds(t * SCHUNK, SCHUNK)], src_v)
        pltpu.sync_copy(dst_hbm.at[c, s, pl.ds(t * SCHUNK, SCHUNK)], dst_v)
        gd = [None, None]
        sd = [None, None]

        def gather(j, p):
            return [pltpu.async_copy(
                        g_hbm.at[src_v.at[j, pl.ds(h * GS, GS)]],
                        bufs[p].at[pl.ds(h * GS, GS), :], semg[p])
                    for h in range(GH)]

        for j in range(SCHUNK):
            p = j & 1
            if sd[p] is not None:
                sd[p].wait()
            gd[p] = gather(j, p)
            if j >= 1:
                q = 1 - p
                for d in gd[q]:
                    d.wait()
                sd[q] = pltpu.async_copy(bufs[q], acc.at[dst_v.at[j - 1]],
                                         sems[q], add=True)
        q = (SCHUNK - 1) & 1
        for d in gd[q]:
            d.wait()
        sd[q] = pltpu.async_copy(bufs[q], acc.at[dst_v.at[SCHUNK - 1]],
                                 sems[q], add=True)
        for p in range(2):
            if sd[p] is not None:
                sd[p].wait()
        return carry

    nstage = jnp.where(c == 0, NCH0 // SCHUNK, NCH1 // SCHUNK)
    lax.fori_loop(0, nstage, stage, 0)
    plsc.subcore_barrier()
    _write_out_slice(buf0, acc, out_sum, c, s)


def _sc_cnt_body(dst_hbm, out_cnt, dst_v, ones_v, cac, sem):
    c = lax.axis_index("c")
    s = lax.axis_index("s")

    _zero_buf(ones_v)
    _zero_acc_slice(ones_v, cac, s)

    ov = jnp.ones((16,), jnp.float32)

    def fo(i, carry):
        ones_v[i // (C // 16), pl.ds((i % (C // 16)) * 16, 16)] = ov
        return carry

    lax.fori_loop(0, K * (C // 16), fo, 0)
    plsc.subcore_barrier()

    def stage(t, carry):
        pltpu.sync_copy(dst_hbm.at[c, s, pl.ds(t * SCHUNK, SCHUNK)], dst_v)
        sds = [pltpu.async_copy(ones_v, cac.at[dst_v.at[j]], sem, add=True)
               for j in range(SCHUNK)]
        for sd in sds:
            sd.wait()
        return carry

    lax.fori_loop(0, NCHUNK // SCHUNK, stage, 0)
    plsc.subcore_barrier()
    _write_out_slice(ones_v, cac, out_cnt, c, s)


def _make_sc_sum():
    return functools.partial(
        pl.kernel,
        out_type=jax.ShapeDtypeStruct((NC, ACC_ROWS, C), jnp.float32),
        mesh=plsc.VectorSubcoreMesh(core_axis_name="c", subcore_axis_name="s",
                                    num_cores=NC, num_subcores=NS),
        scratch_types=[
            pltpu.VMEM((SCHUNK, K), jnp.int32),
            pltpu.VMEM((SCHUNK, K), jnp.int32),
            pltpu.VMEM((K, C), jnp.float32),
            pltpu.VMEM((K, C), jnp.float32),
            pltpu.VMEM_SHARED((ACC_ROWS, C), jnp.float32),
            pltpu.SemaphoreType.DMA,
            pltpu.SemaphoreType.DMA,
            pltpu.SemaphoreType.DMA,
            pltpu.SemaphoreType.DMA,
        ],
    )(_sc_sum_body)


def _make_sc_cnt():
    return functools.partial(
        pl.kernel,
        out_type=jax.ShapeDtypeStruct((NC, ACC_ROWS, C), jnp.float32),
        mesh=plsc.VectorSubcoreMesh(core_axis_name="c", subcore_axis_name="s",
                                    num_cores=NC, num_subcores=NS),
        scratch_types=[
            pltpu.VMEM((SCHUNK, K), jnp.int32),
            pltpu.VMEM((K, C), jnp.float32),
            pltpu.VMEM_SHARED((ACC_ROWS, C), jnp.float32),
            pltpu.SemaphoreType.DMA,
        ],
    )(_sc_cnt_body)


def kernel(x, edge_index, W_embed, b_embed, W1, b1, W2, b2):
    x = x.astype(jnp.float32)

    g = pl.pallas_call(
        _dense_a_body,
        grid=(N // RB,),
        in_specs=[
            pl.BlockSpec((RB, x.shape[1]), lambda i: (i, 0)),
            pl.BlockSpec(W_embed.shape, lambda i: (0, 0)),
            pl.BlockSpec((1, C), lambda i: (0, 0)),
            pl.BlockSpec(W1.shape, lambda i: (0, 0)),
            pl.BlockSpec((1, C), lambda i: (0, 0)),
        ],
        out_specs=pl.BlockSpec((RB, C), lambda i: (i, 0)),
        out_shape=jax.ShapeDtypeStruct((N, C), jnp.float32),
    )(x, W_embed, b_embed.reshape(1, C), W1, b1.reshape(1, C))

    ei = edge_index.astype(jnp.int32)
    pad = EPAD - E
    src_p = jnp.concatenate([ei[0], jnp.zeros((pad,), jnp.int32)])
    dst_p = jnp.concatenate([ei[1], jnp.full((pad,), N, jnp.int32)])
    dst_r = dst_p.reshape(NC, NS, NCHUNK, K)

    e0 = NS * NCH0 * K
    cpad = ((0, 0), (0, 0), (0, NCHMAX - NCH0), (0, 0))
    src_a = jnp.concatenate(
        [jnp.pad(src_p[:e0].reshape(1, NS, NCH0, K), cpad),
         src_p[e0:].reshape(1, NS, NCH1, K)], axis=0)
    dst_a = jnp.concatenate(
        [jnp.pad(dst_p[:e0].reshape(1, NS, NCH0, K), cpad, constant_values=N),
         dst_p[e0:].reshape(1, NS, NCH1, K)], axis=0)

    g2 = jnp.concatenate([g, g], axis=0)
    src_a = src_a + (jnp.arange(NC, dtype=jnp.int32) * N).reshape(NC, 1, 1, 1)

    sums = _make_sc_sum()(g2, src_a, dst_a)
    cnts = _make_sc_cnt()(dst_r)

    cnt = cnts.sum(axis=0)[:, 0:1]

    out2d = pl.pallas_call(
        _dense_b_body,
        grid=(ACC_ROWS // RB2,),
        in_specs=[
            pl.BlockSpec((NC, RB2, C), lambda i: (0, i, 0)),
            pl.BlockSpec((RB2, 1), lambda i: (i, 0)),
            pl.BlockSpec(W2.shape, lambda i: (0, 0)),
            pl.BlockSpec((1, C), lambda i: (0, 0)),
        ],
        out_specs=pl.BlockSpec((RB2, 1), lambda i: (i, 0)),
        out_shape=jax.ShapeDtypeStruct((ACC_ROWS, 1), jnp.float32),
    )(sums, cnt, W2, b2.reshape(1, C))

    return out2d.reshape(ACC_ROWS)[:N]
